# baseline (speedup 1.0000x reference)
import functools

import jax
import jax.numpy as jnp
from jax import lax
from jax.experimental import pallas as pl
from jax.experimental.pallas import tpu as pltpu
from jax.experimental.pallas import tpu_sc as plsc

NUM_ITEMS = 1000000
EMBED_DIM = 32
NC, NS = 2, 16
NW = NC * NS
IDX_MINOR = 128
ROWS_PER_CHUNK = 1024
K = ROWS_PER_CHUNK // IDX_MINOR
LANES = 16
VSTEPS = ROWS_PER_CHUNK // LANES


def _body(n_chunks, idx_hbm, ktab_hbm, etab_hbm, kout_hbm, eout_hbm,
          idx_v, idxh_v, krows_v, epair_v, evals_v, etab_sh,
          sem_i0, sem_i1, sem_k0, sem_k1, sem_e0, sem_e1, sem_w0, sem_w1):
    wid = lax.axis_index("s") * NC + lax.axis_index("c")
    base_row = wid * (n_chunks * ROWS_PER_CHUNK)
    sem_i = (sem_i0, sem_i1)
    sem_k = (sem_k0, sem_k1)
    sem_e = (sem_e0, sem_e1)
    sem_w = (sem_w0, sem_w1)

    @pl.when(lax.axis_index("s") == 0)
    def _():
        pltpu.sync_copy(etab_hbm, etab_sh)
    plsc.subcore_barrier()

    def rows(c):
        return pl.multiple_of(base_row + c * ROWS_PER_CHUNK, ROWS_PER_CHUNK)

    def fire_idx(c, b):
        irow0 = pl.multiple_of(rows(c) // IDX_MINOR, K)
        pltpu.async_copy(idx_hbm.at[pl.ds(irow0, K)], idx_v.at[b], sem_i[b])

    def wait_idx(b):
        pltpu.make_async_copy(
            idx_hbm.at[pl.ds(0, K)], idx_v.at[b], sem_i[b]).wait()

    def halve_idx(b):
        src = idx_v.at[b]
        dst = idxh_v.at[b]
        def step(t, _):
            for u in range(4):
                g = t * 4 + u
                j = g // (IDX_MINOR // LANES)
                o = (g % (IDX_MINOR // LANES)) * LANES
                v = src.at[j][pl.ds(o, LANES)]
                dst.at[j][pl.ds(o, LANES)] = lax.shift_right_logical(v, 1)
            return ()
        lax.fori_loop(0, VSTEPS // 4, step, ())

    def fire_k_gathers(b):
        for j in range(K):
            pltpu.async_copy(
                ktab_hbm.at[idx_v.at[b].at[j]],
                krows_v.at[b].at[pl.ds(j * IDX_MINOR, IDX_MINOR)],
                sem_k[b],
            )

    def fire_e_gathers(b):
        for j in range(K):
            pltpu.async_copy(
                etab_sh.at[idxh_v.at[b].at[j]],
                epair_v.at[b].at[pl.ds(j * IDX_MINOR, IDX_MINOR)],
                sem_e[b],
            )

    def wait_gathers(b):
        pltpu.make_async_copy(
            ktab_hbm.at[pl.ds(0, ROWS_PER_CHUNK)], krows_v.at[b],
            sem_k[b]).wait()
        pltpu.make_async_copy(
            etab_sh.at[pl.ds(0, ROWS_PER_CHUNK)], epair_v.at[b],
            sem_e[b]).wait()

    def unpack_evals(b):
        def step(t, _):
            for u in range(4):
                g = t * 4 + u
                j = g // (IDX_MINOR // LANES)
                o = (g % (IDX_MINOR // LANES)) * LANES
                flat = g * LANES
                pair = epair_v.at[b][pl.ds(flat, LANES)]
                ivec = idx_v.at[b].at[j][pl.ds(o, LANES)]
                odd = ivec & 1
                sh = (1 - odd) << 4
                bits = (pair << sh) & jnp.int32(-65536)
                evals_v.at[b][pl.ds(flat, LANES)] = plsc.bitcast(
                    bits, jnp.float32)
            return ()
        lax.fori_loop(0, VSTEPS // 4, step, ())

    def fire_wb_k(c, b):
        pltpu.async_copy(
            krows_v.at[b], kout_hbm.at[pl.ds(rows(c), ROWS_PER_CHUNK)],
            sem_w[b])

    def fire_wb_e(c, b):
        pltpu.async_copy(
            evals_v.at[b], eout_hbm.at[pl.ds(rows(c), ROWS_PER_CHUNK)],
            sem_w[b])

    def wait_wb(b):
        pltpu.make_async_copy(
            krows_v.at[b], kout_hbm.at[pl.ds(0, ROWS_PER_CHUNK)],
            sem_w[b]).wait()
        pltpu.make_async_copy(
            evals_v.at[b], eout_hbm.at[pl.ds(0, ROWS_PER_CHUNK)],
            sem_w[b]).wait()

    def finish_chunk(c, b):
        wait_gathers(b)
        fire_wb_k(c, b)
        unpack_evals(b)
        fire_wb_e(c, b)

    def start_chunk(b):
        wait_idx(b)
        wait_wb(b)
        fire_k_gathers(b)
        halve_idx(b)
        fire_e_gathers(b)

    fire_idx(0, 0)
    wait_idx(0)
    fire_k_gathers(0)
    halve_idx(0)
    fire_e_gathers(0)
    fire_idx(1, 1)
    wait_idx(1)
    fire_k_gathers(1)
    halve_idx(1)
    fire_e_gathers(1)
    finish_chunk(0, 0)
    fire_idx(2, 0)

    def step(h, _):
        c0 = 2 * h
        c1 = c0 + 1
        finish_chunk(c1 - 2, 1)
        fire_idx(c1, 1)
        start_chunk(0)
        finish_chunk(c0, 0)
        fire_idx(jnp.minimum(c0 + 2, n_chunks - 1), 0)
        start_chunk(1)
        return ()

    lax.fori_loop(1, n_chunks // 2, step, ())

    finish_chunk(n_chunks - 1, 1)
    wait_idx(0)
    wait_wb(0)
    wait_wb(1)


def kernel(item_inputs, k_difficulty, e_discrimination):
    bsz, hist = item_inputs.shape
    b_total = bsz * hist
    assert b_total % (NW * ROWS_PER_CHUNK) == 0
    n_chunks = b_total // (NW * ROWS_PER_CHUNK)
    assert n_chunks % 2 == 0 and n_chunks >= 4
    idx2d = item_inputs.reshape(b_total // IDX_MINOR, IDX_MINOR)
    ebf = e_discrimination.reshape(NUM_ITEMS // 2, 2).astype(jnp.bfloat16)
    epacked = lax.bitcast_convert_type(ebf, jnp.int32)

    mesh = plsc.VectorSubcoreMesh(core_axis_name="c", subcore_axis_name="s")
    run = pl.kernel(
        functools.partial(_body, n_chunks),
        out_type=(
            jax.ShapeDtypeStruct((b_total, EMBED_DIM), jnp.float32),
            jax.ShapeDtypeStruct((b_total,), jnp.float32),
        ),
        mesh=mesh,
        compiler_params=pltpu.CompilerParams(use_tc_tiling_on_sc=False, needs_layout_passes=False),
        scratch_types=[
            pltpu.VMEM((2, K, IDX_MINOR), jnp.int32),
            pltpu.VMEM((2, K, IDX_MINOR), jnp.int32),
            pltpu.VMEM((2, ROWS_PER_CHUNK, EMBED_DIM), jnp.float32),
            pltpu.VMEM((2, ROWS_PER_CHUNK), jnp.int32),
            pltpu.VMEM((2, ROWS_PER_CHUNK), jnp.float32),
            pltpu.VMEM_SHARED((NUM_ITEMS // 2,), jnp.int32),
        ] + [pltpu.SemaphoreType.DMA] * 8,
    )
    kout, eout = run(idx2d, k_difficulty, epacked)
    return (kout.reshape(bsz, hist, EMBED_DIM), eout.reshape(bsz, hist, 1))


# speedup vs baseline: 1.1157x; 1.1157x over previous
import functools

import jax
import jax.numpy as jnp
from jax import lax
from jax.experimental import pallas as pl
from jax.experimental.pallas import tpu as pltpu
from jax.experimental.pallas import tpu_sc as plsc

EMBED_DIM = 32
NC, NS = 2, 16
NW = NC * NS
IDX_MINOR = 256
ROWS_PER_CHUNK = 1024
K = ROWS_PER_CHUNK // IDX_MINOR
NBUF = 3


def _body(n_chunks, idx_hbm, ktab_hbm, etab_hbm, kout_hbm, eout_hbm,
          idx_v, krows_v, evals_v, *sems):
    wid = lax.axis_index("s") * NC + lax.axis_index("c")
    base_row = wid * (n_chunks * ROWS_PER_CHUNK)
    sem_i = sems[0:NBUF]
    sem_k = sems[NBUF:2 * NBUF]
    sem_e = sems[2 * NBUF:3 * NBUF]
    sem_w = sems[3 * NBUF:4 * NBUF]

    def rows(c):
        return pl.multiple_of(base_row + c * ROWS_PER_CHUNK, ROWS_PER_CHUNK)

    def fire_idx(c, b):
        irow0 = pl.multiple_of(rows(c) // IDX_MINOR, K)
        pltpu.async_copy(idx_hbm.at[pl.ds(irow0, K)], idx_v.at[b], sem_i[b])

    def wait_idx(b):
        pltpu.make_async_copy(
            idx_hbm.at[pl.ds(0, K)], idx_v.at[b], sem_i[b]).wait()

    def fire_gathers(b):
        for j in range(K):
            pltpu.async_copy(
                ktab_hbm.at[idx_v.at[b].at[j]],
                krows_v.at[b].at[pl.ds(j * IDX_MINOR, IDX_MINOR)],
                sem_k[b],
            )
            pltpu.async_copy(
                etab_hbm.at[idx_v.at[b].at[j]],
                evals_v.at[b].at[pl.ds(j * IDX_MINOR, IDX_MINOR)],
                sem_e[b],
            )

    def wait_gathers(b):
        pltpu.make_async_copy(
            ktab_hbm.at[pl.ds(0, ROWS_PER_CHUNK)], krows_v.at[b],
            sem_k[b]).wait()
        pltpu.make_async_copy(
            etab_hbm.at[pl.ds(0, ROWS_PER_CHUNK)], evals_v.at[b],
            sem_e[b]).wait()

    def fire_wb(c, b):
        pltpu.async_copy(
            krows_v.at[b], kout_hbm.at[pl.ds(rows(c), ROWS_PER_CHUNK)],
            sem_w[b])
        pltpu.async_copy(
            evals_v.at[b], eout_hbm.at[pl.ds(rows(c), ROWS_PER_CHUNK)],
            sem_w[b])

    def wait_wb(b):
        pltpu.make_async_copy(
            krows_v.at[b], kout_hbm.at[pl.ds(0, ROWS_PER_CHUNK)],
            sem_w[b]).wait()
        pltpu.make_async_copy(
            evals_v.at[b], eout_hbm.at[pl.ds(0, ROWS_PER_CHUNK)],
            sem_w[b]).wait()

    def finish(c, b):
        wait_gathers(b)
        fire_wb(c, b)

    def slot(c, b, skip_wb_wait=False):
        finish(c - 2, (b + 1) % NBUF)
        cn = jnp.minimum(c + 1, n_chunks - 1)
        fire_idx(cn, (b + 1) % NBUF)
        wait_idx(b)
        if not skip_wb_wait:
            wait_wb(b)
        fire_gathers(b)

    fire_idx(0, 0)
    wait_idx(0)
    fire_gathers(0)
    fire_idx(1, 1)
    wait_idx(1)
    fire_gathers(1)
    fire_idx(2, 2)
    slot(2, 2, skip_wb_wait=True)
    slot(3, 0)
    slot(4, 1)

    def step(h, _):
        c = 3 * h + 5
        slot(c, 2)
        slot(c + 1, 0)
        slot(c + 2, 1)
        return ()

    lax.fori_loop(0, (n_chunks - 5) // 3, step, ())

    slot(n_chunks - 2, 2)
    slot(n_chunks - 1, 0)
    finish(n_chunks - 2, 2)
    finish(n_chunks - 1, 0)
    wait_idx(1)
    wait_wb(1)
    wait_wb(2)
    wait_wb(0)


def kernel(item_inputs, k_difficulty, e_discrimination):
    bsz, hist = item_inputs.shape
    b_total = bsz * hist
    assert b_total % (NW * ROWS_PER_CHUNK) == 0
    n_chunks = b_total // (NW * ROWS_PER_CHUNK)
    assert n_chunks >= 8 and (n_chunks - 7) % 3 == 0
    idx2d = item_inputs.reshape(b_total // IDX_MINOR, IDX_MINOR)
    etab = e_discrimination.reshape(-1)

    mesh = plsc.VectorSubcoreMesh(core_axis_name="c", subcore_axis_name="s")
    run = pl.kernel(
        functools.partial(_body, n_chunks),
        out_type=(
            jax.ShapeDtypeStruct((b_total, EMBED_DIM), jnp.float32),
            jax.ShapeDtypeStruct((b_total,), jnp.float32),
        ),
        mesh=mesh,
        compiler_params=pltpu.CompilerParams(use_tc_tiling_on_sc=False),
        scratch_types=[
            pltpu.VMEM((NBUF, K, IDX_MINOR), jnp.int32),
            pltpu.VMEM((NBUF, ROWS_PER_CHUNK, EMBED_DIM), jnp.float32),
            pltpu.VMEM((NBUF, ROWS_PER_CHUNK), jnp.float32),
        ] + [pltpu.SemaphoreType.DMA] * (4 * NBUF),
    )
    kout, eout = run(idx2d, k_difficulty, etab)
    return (kout.reshape(bsz, hist, EMBED_DIM), eout.reshape(bsz, hist, 1))


# speedup vs baseline: 1.1158x; 1.0001x over previous
import functools

import jax
import jax.numpy as jnp
from jax import lax
from jax.experimental import pallas as pl
from jax.experimental.pallas import tpu as pltpu
from jax.experimental.pallas import tpu_sc as plsc

EMBED_DIM = 32
NC, NS = 2, 16
NW = NC * NS
IDX_MINOR = 128
ROWS_PER_CHUNK = 1024
K = ROWS_PER_CHUNK // IDX_MINOR
NBUF = 3


def _body(n_chunks, idx_hbm, ktab_hbm, etab_hbm, kout_hbm, eout_hbm,
          idx_v, krows_v, evals_v, *sems):
    wid = lax.axis_index("s") * NC + lax.axis_index("c")
    base_row = wid * (n_chunks * ROWS_PER_CHUNK)
    sem_i = sems[0:NBUF]
    sem_k = sems[NBUF:2 * NBUF]
    sem_e = sems[2 * NBUF:3 * NBUF]
    sem_w = sems[3 * NBUF:4 * NBUF]

    def rows(c):
        return pl.multiple_of(base_row + c * ROWS_PER_CHUNK, ROWS_PER_CHUNK)

    def fire_idx(c, b):
        irow0 = pl.multiple_of(rows(c) // IDX_MINOR, K)
        pltpu.async_copy(idx_hbm.at[pl.ds(irow0, K)], idx_v.at[b], sem_i[b])

    def wait_idx(b):
        pltpu.make_async_copy(
            idx_hbm.at[pl.ds(0, K)], idx_v.at[b], sem_i[b]).wait()

    def fire_gathers(b):
        for j in range(K):
            pltpu.async_copy(
                ktab_hbm.at[idx_v.at[b].at[j]],
                krows_v.at[b].at[pl.ds(j * IDX_MINOR, IDX_MINOR)],
                sem_k[b],
            )
            pltpu.async_copy(
                etab_hbm.at[idx_v.at[b].at[j]],
                evals_v.at[b].at[pl.ds(j * IDX_MINOR, IDX_MINOR)],
                sem_e[b],
            )

    def wait_gathers(b):
        pltpu.make_async_copy(
            ktab_hbm.at[pl.ds(0, ROWS_PER_CHUNK)], krows_v.at[b],
            sem_k[b]).wait()
        pltpu.make_async_copy(
            etab_hbm.at[pl.ds(0, ROWS_PER_CHUNK)], evals_v.at[b],
            sem_e[b]).wait()

    def fire_wb(c, b):
        pltpu.async_copy(
            krows_v.at[b], kout_hbm.at[pl.ds(rows(c), ROWS_PER_CHUNK)],
            sem_w[b])
        pltpu.async_copy(
            evals_v.at[b], eout_hbm.at[pl.ds(rows(c), ROWS_PER_CHUNK)],
            sem_w[b])

    def wait_wb(b):
        pltpu.make_async_copy(
            krows_v.at[b], kout_hbm.at[pl.ds(0, ROWS_PER_CHUNK)],
            sem_w[b]).wait()
        pltpu.make_async_copy(
            evals_v.at[b], eout_hbm.at[pl.ds(0, ROWS_PER_CHUNK)],
            sem_w[b]).wait()

    def finish(c, b):
        wait_gathers(b)
        fire_wb(c, b)

    def slot(c, b, skip_wb_wait=False):
        finish(c - 2, (b + 1) % NBUF)
        cn = jnp.minimum(c + 1, n_chunks - 1)
        fire_idx(cn, (b + 1) % NBUF)
        wait_idx(b)
        if not skip_wb_wait:
            wait_wb(b)
        fire_gathers(b)

    fire_idx(0, 0)
    wait_idx(0)
    fire_gathers(0)
    fire_idx(1, 1)
    wait_idx(1)
    fire_gathers(1)
    fire_idx(2, 2)
    slot(2, 2, skip_wb_wait=True)
    slot(3, 0)
    slot(4, 1)

    def step(h, _):
        c = 3 * h + 5
        slot(c, 2)
        slot(c + 1, 0)
        slot(c + 2, 1)
        return ()

    lax.fori_loop(0, (n_chunks - 5) // 3, step, ())

    slot(n_chunks - 2, 2)
    slot(n_chunks - 1, 0)
    finish(n_chunks - 2, 2)
    finish(n_chunks - 1, 0)
    wait_idx(1)
    wait_wb(1)
    wait_wb(2)
    wait_wb(0)


def kernel(item_inputs, k_difficulty, e_discrimination):
    bsz, hist = item_inputs.shape
    b_total = bsz * hist
    assert b_total % (NW * ROWS_PER_CHUNK) == 0
    n_chunks = b_total // (NW * ROWS_PER_CHUNK)
    assert n_chunks >= 8 and (n_chunks - 7) % 3 == 0
    idx2d = item_inputs.reshape(b_total // IDX_MINOR, IDX_MINOR)
    etab = e_discrimination.reshape(-1)

    mesh = plsc.VectorSubcoreMesh(core_axis_name="c", subcore_axis_name="s")
    run = pl.kernel(
        functools.partial(_body, n_chunks),
        out_type=(
            jax.ShapeDtypeStruct((b_total, EMBED_DIM), jnp.float32),
            jax.ShapeDtypeStruct((b_total,), jnp.float32),
        ),
        mesh=mesh,
        compiler_params=pltpu.CompilerParams(use_tc_tiling_on_sc=False),
        scratch_types=[
            pltpu.VMEM((NBUF, K, IDX_MINOR), jnp.int32),
            pltpu.VMEM((NBUF, ROWS_PER_CHUNK, EMBED_DIM), jnp.float32),
            pltpu.VMEM((NBUF, ROWS_PER_CHUNK), jnp.float32),
        ] + [pltpu.SemaphoreType.DMA] * (4 * NBUF),
    )
    kout, eout = run(idx2d, k_difficulty, etab)
    return (kout.reshape(bsz, hist, EMBED_DIM), eout.reshape(bsz, hist, 1))
